# baseline (speedup 1.0000x reference)
import functools

import jax
import jax.numpy as jnp
from jax.experimental import pallas as pl
from jax.experimental.pallas import tpu as pltpu


def _layer_body(mat_ref, src_ref, w_ref, b_ref, out_ref, s_ref, *, relu):
    @pl.when(pl.program_id(0) == 0)
    def _():
        s_ref[...] = jnp.dot(src_ref[...], w_ref[...],
                             preferred_element_type=jnp.float32)
    acc = jnp.dot(mat_ref[...], s_ref[...],
                  preferred_element_type=jnp.float32)
    acc = acc + b_ref[...]
    if relu:
        acc = jnp.maximum(acc, 0.0)
    out_ref[...] = acc


def _layer(mat, src, w, b, *, relu, block_rows):
    rows, k = mat.shape
    kf, f = src.shape
    c = w.shape[1]
    grid = (rows // block_rows,)
    return pl.pallas_call(
        functools.partial(_layer_body, relu=relu),
        grid=grid,
        in_specs=[
            pl.BlockSpec((block_rows, k), lambda i: (i, 0)),
            pl.BlockSpec((kf, f), lambda i: (0, 0)),
            pl.BlockSpec((f, c), lambda i: (0, 0)),
            pl.BlockSpec((1, c), lambda i: (0, 0)),
        ],
        out_specs=pl.BlockSpec((block_rows, c), lambda i: (i, 0)),
        out_shape=jax.ShapeDtypeStruct((rows, c), jnp.float32),
        scratch_shapes=[pltpu.VMEM((kf, c), jnp.float32)],
        compiler_params=pltpu.CompilerParams(
            dimension_semantics=("arbitrary",),
        ),
    )(mat, src, w, b)


def kernel(x, NF, FN, W1, b1, W2, b2):
    b1r = b1.reshape(1, -1)
    b2r = b2.reshape(1, -1)
    h = _layer(FN, x, W1, b1r, relu=True, block_rows=512)
    out = _layer(NF, h, W2, b2r, relu=False, block_rows=512)
    return out


# speedup vs baseline: 1.0131x; 1.0131x over previous
import functools

import jax
import jax.numpy as jnp
from jax.experimental import pallas as pl
from jax.experimental.pallas import tpu as pltpu


def _layer_body(mat_ref, src_ref, w_ref, b_ref, out_ref, s_ref, *, relu):
    @pl.when(pl.program_id(0) == 0)
    def _():
        s_ref[...] = jnp.dot(src_ref[...], w_ref[...],
                             preferred_element_type=jnp.float32)
    acc = jnp.dot(mat_ref[...], s_ref[...],
                  preferred_element_type=jnp.float32)
    acc = acc + b_ref[...]
    if relu:
        acc = jnp.maximum(acc, 0.0)
    out_ref[...] = acc


def _layer(mat, src, w, b, *, relu, block_rows):
    rows, k = mat.shape
    kf, f = src.shape
    c = w.shape[1]
    grid = (rows // block_rows,)
    return pl.pallas_call(
        functools.partial(_layer_body, relu=relu),
        grid=grid,
        in_specs=[
            pl.BlockSpec((block_rows, k), lambda i: (i, 0)),
            pl.BlockSpec((kf, f), lambda i: (0, 0)),
            pl.BlockSpec((f, c), lambda i: (0, 0)),
            pl.BlockSpec((1, c), lambda i: (0, 0)),
        ],
        out_specs=pl.BlockSpec((block_rows, c), lambda i: (i, 0)),
        out_shape=jax.ShapeDtypeStruct((rows, c), jnp.float32),
        scratch_shapes=[pltpu.VMEM((kf, c), jnp.float32)],
        compiler_params=pltpu.CompilerParams(
            dimension_semantics=("arbitrary",),
        ),
    )(mat, src, w, b)


def kernel(x, NF, FN, W1, b1, W2, b2):
    b1r = b1.reshape(1, -1)
    b2r = b2.reshape(1, -1)
    h = _layer(FN, x, W1, b1r, relu=True, block_rows=256)
    out = _layer(NF, h, W2, b2r, relu=False, block_rows=256)
    return out


# speedup vs baseline: 1.0142x; 1.0011x over previous
import functools

import jax
import jax.numpy as jnp
from jax.experimental import pallas as pl
from jax.experimental.pallas import tpu as pltpu


def _layer_body(mat_ref, src_ref, w_ref, b_ref, out_ref, s_ref, *, relu):
    @pl.when(pl.program_id(0) == 0)
    def _():
        s_ref[...] = jnp.dot(src_ref[...], w_ref[...],
                             preferred_element_type=jnp.float32
                             ).astype(jnp.bfloat16)
    acc = jnp.dot(mat_ref[...].astype(jnp.bfloat16), s_ref[...],
                  preferred_element_type=jnp.float32)
    acc = acc + b_ref[...]
    if relu:
        acc = jnp.maximum(acc, 0.0)
    out_ref[...] = acc


def _layer(mat, src, w, b, *, relu, block_rows):
    rows, k = mat.shape
    kf, f = src.shape
    c = w.shape[1]
    grid = (rows // block_rows,)
    return pl.pallas_call(
        functools.partial(_layer_body, relu=relu),
        grid=grid,
        in_specs=[
            pl.BlockSpec((block_rows, k), lambda i: (i, 0)),
            pl.BlockSpec((kf, f), lambda i: (0, 0)),
            pl.BlockSpec((f, c), lambda i: (0, 0)),
            pl.BlockSpec((1, c), lambda i: (0, 0)),
        ],
        out_specs=pl.BlockSpec((block_rows, c), lambda i: (i, 0)),
        out_shape=jax.ShapeDtypeStruct((rows, c), jnp.float32),
        scratch_shapes=[pltpu.VMEM((kf, c), jnp.bfloat16)],
        compiler_params=pltpu.CompilerParams(
            dimension_semantics=("arbitrary",),
        ),
    )(mat, src, w, b)


def kernel(x, NF, FN, W1, b1, W2, b2):
    b1r = b1.reshape(1, -1)
    b2r = b2.reshape(1, -1)
    h = _layer(FN, x, W1, b1r, relu=True, block_rows=256)
    out = _layer(NF, h, W2, b2r, relu=False, block_rows=256)
    return out
